# baseline (speedup 1.0000x reference)
import functools

import jax
import jax.numpy as jnp
from jax import lax
from jax.experimental import pallas as pl
from jax.experimental.pallas import tpu as pltpu
from jax.experimental.pallas import tpu_sc as plsc


def _proj_body(x_ref, w_ref, al_ref, ar_ref,
               feat_ref, el_ref, er_ref, elm_ref, erm_ref, bound_ref):
    i = pl.program_id(0)
    feat = jnp.dot(x_ref[...], w_ref[...], preferred_element_type=jnp.float32)
    feat_ref[...] = feat
    el = jnp.sum(feat * al_ref[...], axis=1, keepdims=True)
    er = jnp.sum(feat * ar_ref[...], axis=1, keepdims=True)
    el_ref[...] = el
    er_ref[...] = er
    bl = jnp.max(el)
    br = jnp.max(er)

    @pl.when(i == 0)
    def _():
        elm_ref[0, 0] = bl
        erm_ref[0, 0] = br

    @pl.when(i > 0)
    def _():
        elm_ref[0, 0] = jnp.maximum(elm_ref[0, 0], bl)
        erm_ref[0, 0] = jnp.maximum(erm_ref[0, 0], br)

    @pl.when(i == pl.num_programs(0) - 1)
    def _():
        m = elm_ref[0, 0] + erm_ref[0, 0]
        bound_ref[...] = jnp.full((1, _LANES), jnp.maximum(m, 0.2 * m),
                                  jnp.float32)


def _project(x, W, attn_l, attn_r):
    n, d_in = x.shape
    d_out = W.shape[1]
    bn = 2000
    nblk = n // bn
    smem_scalar = pl.BlockSpec(memory_space=pltpu.SMEM)
    return pl.pallas_call(
        _proj_body,
        grid=(nblk,),
        in_specs=[
            pl.BlockSpec((bn, d_in), lambda i: (i, 0)),
            pl.BlockSpec((d_in, d_out), lambda i: (0, 0)),
            pl.BlockSpec((1, d_out), lambda i: (0, 0)),
            pl.BlockSpec((1, d_out), lambda i: (0, 0)),
        ],
        out_specs=[
            pl.BlockSpec((bn, d_out), lambda i: (i, 0)),
            pl.BlockSpec((bn, 1), lambda i: (i, 0)),
            pl.BlockSpec((bn, 1), lambda i: (i, 0)),
            smem_scalar,
            smem_scalar,
            pl.BlockSpec((1, _LANES), lambda i: (0, 0)),
        ],
        out_shape=[
            jax.ShapeDtypeStruct((n, d_out), jnp.float32),
            jax.ShapeDtypeStruct((n, 1), jnp.float32),
            jax.ShapeDtypeStruct((n, 1), jnp.float32),
            jax.ShapeDtypeStruct((1, 1), jnp.float32),
            jax.ShapeDtypeStruct((1, 1), jnp.float32),
            jax.ShapeDtypeStruct((1, _LANES), jnp.float32),
        ],
    )(x, W, attn_l, attn_r)


_CHUNK = 80
_LANES = 16


def _make_sc_edge_kernel(n, d, e, num_cores, num_subcores):
    tiles = num_cores * num_subcores
    ept = e // tiles
    nchunk = ept // _CHUNK
    assert ept % _CHUNK == 0 and e % tiles == 0
    slab = 1000
    nslab = n // slab
    assert n % slab == 0 and slab % 8 == 0
    mesh = plsc.VectorSubcoreMesh(
        core_axis_name="c", subcore_axis_name="s",
        num_cores=num_cores, num_subcores=num_subcores)

    @functools.partial(
        pl.kernel,
        out_type=[
            jax.ShapeDtypeStruct((num_cores, n, d), jnp.float32),
            jax.ShapeDtypeStruct((num_cores, n), jnp.float32),
        ],
        mesh=mesh,
        scratch_types=[
            pltpu.VMEM_SHARED((n, d), jnp.float32),
            pltpu.VMEM_SHARED((n,), jnp.float32),
            pltpu.VMEM((n,), jnp.float32),
            pltpu.VMEM((n,), jnp.float32),
            pltpu.VMEM((_LANES,), jnp.float32),
            pltpu.VMEM((3, _CHUNK), jnp.int32),
            pltpu.VMEM((3, _CHUNK), jnp.int32),
            pltpu.VMEM((2, _CHUNK), jnp.int32),
            pltpu.VMEM((2, _CHUNK), jnp.float32),
            pltpu.VMEM((2, _CHUNK, d), jnp.float32),
            pltpu.SemaphoreType.DMA((2,)),
            pltpu.SemaphoreType.DMA((2,)),
            pltpu.SemaphoreType.DMA((3,)),
            pltpu.SemaphoreType.DMA((2,)),
        ],
        compiler_params=pltpu.CompilerParams(needs_layout_passes=False),
    )
    def sc_edge_kernel(feat_hbm, el_hbm, er_hbm, eidx_hbm, bound_hbm,
                       zrows_hbm, zvec_hbm,
                       acc_out, den_out,
                       acc_sh, den_sh, el_v, er_v, bound_v,
                       src_c, dst_c, dst_cw, w_v, rows2,
                       gsem, ssem, isem, wsem):
        cid = lax.axis_index("c")
        sid = lax.axis_index("s")
        wid = cid * num_subcores + sid

        pltpu.sync_copy(el_hbm, el_v)
        pltpu.sync_copy(er_hbm, er_v)
        pltpu.sync_copy(bound_hbm.at[0], bound_v)

        @pl.when(sid < nslab)
        def _():
            pltpu.sync_copy(zrows_hbm.at[pl.ds(sid * slab, slab)],
                            acc_sh.at[pl.ds(sid * slab, slab)])

        @pl.when(sid == 0)
        def _():
            pltpu.sync_copy(zvec_hbm, den_sh)

        plsc.subcore_barrier()

        bvec = bound_v[...]

        def fetch_idx(ci, b3):
            pltpu.async_copy(eidx_hbm.at[wid, ci], src_c.at[b3], isem.at[b3])
            pltpu.async_copy(eidx_hbm.at[tiles + wid, ci], dst_c.at[b3], isem.at[b3])

        def wait_idx(ci, b3):
            pltpu.make_async_copy(eidx_hbm.at[wid, ci], src_c.at[b3],
                                  isem.at[b3]).wait()
            pltpu.make_async_copy(eidx_hbm.at[tiles + wid, ci], dst_c.at[b3],
                                  isem.at[b3]).wait()

        def gather(ci, b3, buf):
            pltpu.async_copy(feat_hbm.at[src_c.at[b3]], rows2.at[buf],
                             gsem.at[buf])

        def scat_wait(b3, buf):
            pltpu.make_async_copy(rows2.at[buf], acc_sh.at[dst_c.at[b3]],
                                  ssem.at[buf]).wait()

        def wscat_wait(buf):
            pltpu.make_async_copy(w_v.at[buf], den_sh.at[dst_cw.at[buf]],
                                  wsem.at[buf]).wait()

        pltpu.sync_copy(eidx_hbm.at[wid, 0], src_c.at[0])
        pltpu.sync_copy(eidx_hbm.at[tiles + wid, 0], dst_c.at[0])
        gather(0, 0, 0)

        @pl.when(jnp.int32(nchunk) > 1)
        def _():
            fetch_idx(1, 1)

        def chunk_body(ci, carry):
            cur = lax.rem(ci, 2)
            nxt = lax.rem(ci + 1, 2)
            i3 = lax.rem(ci, 3)
            n3 = lax.rem(ci + 1, 3)
            f3 = lax.rem(ci + 2, 3)

            @pl.when(ci >= 1)
            def _():
                scat_wait(lax.rem(ci - 1, 3), nxt)

            @pl.when(ci + 1 < nchunk)
            def _():
                wait_idx(ci + 1, n3)
                gather(ci + 1, n3, nxt)

            @pl.when(ci + 2 < nchunk)
            def _():
                fetch_idx(ci + 2, f3)

            @pl.when(ci >= 2)
            def _():
                wscat_wait(cur)

            for g in range(_CHUNK // _LANES):
                sl = pl.ds(g * _LANES, _LANES)
                dv = dst_c[i3, sl]
                ee = (plsc.load_gather(el_v, [src_c[i3, sl]])
                      + plsc.load_gather(er_v, [dv]))
                ee = jnp.maximum(ee, 0.2 * ee)
                w_v[cur, sl] = jnp.exp(ee - bvec)
                dst_cw[cur, sl] = dv
            pltpu.async_copy(w_v.at[cur], den_sh.at[dst_cw.at[cur]],
                             wsem.at[cur], add=True)

            pltpu.make_async_copy(feat_hbm.at[src_c.at[i3]], rows2.at[cur],
                                  gsem.at[cur]).wait()

            @plsc.parallel_loop(0, _CHUNK, step=1, unroll=8)
            def _(ei):
                wb = plsc.load_gather(
                    w_v.at[cur], [jnp.full((_LANES,), 0, jnp.int32) + ei])
                for db in range(d // _LANES):
                    sl = pl.ds(db * _LANES, _LANES)
                    rows2[cur, ei, sl] = rows2[cur, ei, sl] * wb
            pltpu.async_copy(rows2.at[cur], acc_sh.at[dst_c.at[i3]],
                             ssem.at[cur], add=True)
            return carry

        lax.fori_loop(0, nchunk, chunk_body, 0, unroll=5)
        scat_wait(lax.rem(nchunk - 1, 3), lax.rem(nchunk - 1, 2))
        wscat_wait(lax.rem(nchunk - 1, 2))

        @pl.when(jnp.int32(nchunk) >= 2)
        def _():
            wscat_wait(lax.rem(nchunk - 2, 2))

        plsc.subcore_barrier()

        @pl.when(sid < nslab)
        def _():
            pltpu.sync_copy(acc_sh.at[pl.ds(sid * slab, slab)],
                            acc_out.at[cid, pl.ds(sid * slab, slab)])

        @pl.when(sid == 0)
        def _():
            pltpu.sync_copy(den_sh, den_out.at[cid])

    return sc_edge_kernel


def _combine_body(acc_ref, den_ref, bias_ref, out_ref):
    a = acc_ref[0] + acc_ref[1]
    dcol = jnp.sum(den_ref[...], axis=0)
    safe = jnp.where(dcol > 0.0, a / jnp.where(dcol > 0.0, dcol, 1.0), 0.0)
    out_ref[...] = jnp.tanh(safe + bias_ref[...])


def _combine(acc, den, bias2d):
    nc, n, d = acc.shape
    nt = den.shape[0]
    bn = 2000
    nblk = n // bn
    return pl.pallas_call(
        _combine_body,
        grid=(nblk,),
        in_specs=[
            pl.BlockSpec((nc, bn, d), lambda i: (0, i, 0)),
            pl.BlockSpec((nt, bn, 1), lambda i: (0, i, 0)),
            pl.BlockSpec((1, d), lambda i: (0, 0)),
        ],
        out_specs=pl.BlockSpec((bn, d), lambda i: (i, 0)),
        out_shape=jax.ShapeDtypeStruct((n, d), jnp.float32),
    )(acc, den.reshape(nt, n, 1), bias2d)


def kernel(x, edge_index, W, attn_l, attn_r, bias):
    n, d_in = x.shape
    d = attn_l.shape[1]
    e = edge_index.shape[1]

    feat, el, er, _elm, _erm, boundv = _project(x, W, attn_l, attn_r)

    info = plsc.get_sparse_core_info()
    tiles = info.num_cores * info.num_subcores
    nchunk = e // (tiles * _CHUNK)
    eidx = edge_index.reshape(2 * tiles, nchunk, _CHUNK)
    zrows = jnp.zeros((n, d), jnp.float32)
    zvec = jnp.zeros((n,), jnp.float32)

    sc_kernel = _make_sc_edge_kernel(n, d, e, info.num_cores, info.num_subcores)
    acc, den = sc_kernel(feat, el.reshape(n), er.reshape(n), eidx,
                         boundv, zrows, zvec)

    return _combine(acc, den, bias.reshape(1, d))


# speedup vs baseline: 1.0262x; 1.0262x over previous
import functools

import jax
import jax.numpy as jnp
from jax import lax
from jax.experimental import pallas as pl
from jax.experimental.pallas import tpu as pltpu
from jax.experimental.pallas import tpu_sc as plsc


def _proj_body(x_ref, w_ref, al_ref, ar_ref,
               feat_ref, el_ref, er_ref, elm_ref, erm_ref, bound_ref):
    i = pl.program_id(0)
    feat = jnp.dot(x_ref[...], w_ref[...], preferred_element_type=jnp.float32)
    feat_ref[...] = feat
    el = jnp.sum(feat * al_ref[...], axis=1, keepdims=True)
    er = jnp.sum(feat * ar_ref[...], axis=1, keepdims=True)
    el_ref[...] = el
    er_ref[...] = er
    bl = jnp.max(el)
    br = jnp.max(er)

    @pl.when(i == 0)
    def _():
        elm_ref[0, 0] = bl
        erm_ref[0, 0] = br

    @pl.when(i > 0)
    def _():
        elm_ref[0, 0] = jnp.maximum(elm_ref[0, 0], bl)
        erm_ref[0, 0] = jnp.maximum(erm_ref[0, 0], br)

    @pl.when(i == pl.num_programs(0) - 1)
    def _():
        m = elm_ref[0, 0] + erm_ref[0, 0]
        bound_ref[...] = jnp.full((1, _LANES), jnp.maximum(m, 0.2 * m),
                                  jnp.float32)


def _project(x, W, attn_l, attn_r):
    n, d_in = x.shape
    d_out = W.shape[1]
    bn = 2000
    nblk = n // bn
    smem_scalar = pl.BlockSpec(memory_space=pltpu.SMEM)
    return pl.pallas_call(
        _proj_body,
        grid=(nblk,),
        in_specs=[
            pl.BlockSpec((bn, d_in), lambda i: (i, 0)),
            pl.BlockSpec((d_in, d_out), lambda i: (0, 0)),
            pl.BlockSpec((1, d_out), lambda i: (0, 0)),
            pl.BlockSpec((1, d_out), lambda i: (0, 0)),
        ],
        out_specs=[
            pl.BlockSpec((bn, d_out), lambda i: (i, 0)),
            pl.BlockSpec((bn, 1), lambda i: (i, 0)),
            pl.BlockSpec((bn, 1), lambda i: (i, 0)),
            smem_scalar,
            smem_scalar,
            pl.BlockSpec((1, _LANES), lambda i: (0, 0)),
        ],
        out_shape=[
            jax.ShapeDtypeStruct((n, d_out), jnp.float32),
            jax.ShapeDtypeStruct((n, 1), jnp.float32),
            jax.ShapeDtypeStruct((n, 1), jnp.float32),
            jax.ShapeDtypeStruct((1, 1), jnp.float32),
            jax.ShapeDtypeStruct((1, 1), jnp.float32),
            jax.ShapeDtypeStruct((1, _LANES), jnp.float32),
        ],
    )(x, W, attn_l, attn_r)


_CHUNK = 80
_LANES = 16


def _make_sc_edge_kernel(n, d, e, num_cores, num_subcores):
    tiles = num_cores * num_subcores
    ept = e // tiles
    nchunk = ept // _CHUNK
    assert ept % _CHUNK == 0 and e % tiles == 0
    slab = 1000
    nslab = n // slab
    assert n % slab == 0 and slab % 8 == 0
    mesh = plsc.VectorSubcoreMesh(
        core_axis_name="c", subcore_axis_name="s",
        num_cores=num_cores, num_subcores=num_subcores)

    @functools.partial(
        pl.kernel,
        out_type=[
            jax.ShapeDtypeStruct((num_cores, n, d), jnp.float32),
            jax.ShapeDtypeStruct((num_cores, n), jnp.float32),
        ],
        mesh=mesh,
        scratch_types=[
            pltpu.VMEM_SHARED((n, d), jnp.float32),
            pltpu.VMEM_SHARED((n,), jnp.float32),
            pltpu.VMEM((n,), jnp.float32),
            pltpu.VMEM((n,), jnp.float32),
            pltpu.VMEM((_LANES,), jnp.float32),
            pltpu.VMEM((3, _CHUNK), jnp.int32),
            pltpu.VMEM((3, _CHUNK), jnp.int32),
            pltpu.VMEM((2, _CHUNK), jnp.int32),
            pltpu.VMEM((2, _CHUNK), jnp.float32),
            pltpu.VMEM((2, _CHUNK, d), jnp.float32),
            pltpu.SemaphoreType.DMA((2,)),
            pltpu.SemaphoreType.DMA((2,)),
            pltpu.SemaphoreType.DMA((3,)),
            pltpu.SemaphoreType.DMA((2,)),
        ],
        compiler_params=pltpu.CompilerParams(needs_layout_passes=False),
    )
    def sc_edge_kernel(feat_hbm, el_hbm, er_hbm, eidx_hbm, bound_hbm,
                       zrows_hbm, zvec_hbm,
                       acc_out, den_out,
                       acc_sh, den_sh, el_v, er_v, bound_v,
                       src_c, dst_c, dst_cw, w_v, rows2,
                       gsem, ssem, isem, wsem):
        cid = lax.axis_index("c")
        sid = lax.axis_index("s")
        wid = cid * num_subcores + sid

        pltpu.sync_copy(el_hbm, el_v)
        pltpu.sync_copy(er_hbm, er_v)
        pltpu.sync_copy(bound_hbm.at[0], bound_v)

        @pl.when(sid < nslab)
        def _():
            pltpu.sync_copy(zrows_hbm.at[pl.ds(sid * slab, slab)],
                            acc_sh.at[pl.ds(sid * slab, slab)])

        @pl.when(sid == 0)
        def _():
            pltpu.sync_copy(zvec_hbm, den_sh)

        plsc.subcore_barrier()

        bvec = bound_v[...]

        def fetch_idx(ci, b3):
            pltpu.async_copy(eidx_hbm.at[wid, ci], src_c.at[b3], isem.at[b3])
            pltpu.async_copy(eidx_hbm.at[tiles + wid, ci], dst_c.at[b3], isem.at[b3])

        def wait_idx(ci, b3):
            pltpu.make_async_copy(eidx_hbm.at[wid, ci], src_c.at[b3],
                                  isem.at[b3]).wait()
            pltpu.make_async_copy(eidx_hbm.at[tiles + wid, ci], dst_c.at[b3],
                                  isem.at[b3]).wait()

        def gather(ci, b3, buf):
            pltpu.async_copy(feat_hbm.at[src_c.at[b3]], rows2.at[buf],
                             gsem.at[buf])

        def scat_wait(b3, buf):
            pltpu.make_async_copy(rows2.at[buf], acc_sh.at[dst_c.at[b3]],
                                  ssem.at[buf]).wait()

        def wscat_wait(buf):
            pltpu.make_async_copy(w_v.at[buf], den_sh.at[dst_cw.at[buf]],
                                  wsem.at[buf]).wait()

        pltpu.sync_copy(eidx_hbm.at[wid, 0], src_c.at[0])
        pltpu.sync_copy(eidx_hbm.at[tiles + wid, 0], dst_c.at[0])
        gather(0, 0, 0)

        @pl.when(jnp.int32(nchunk) > 1)
        def _():
            fetch_idx(1, 1)

        def chunk_body(ci, carry):
            cur = lax.rem(ci, 2)
            nxt = lax.rem(ci + 1, 2)
            i3 = lax.rem(ci, 3)
            n3 = lax.rem(ci + 1, 3)
            f3 = lax.rem(ci + 2, 3)

            @pl.when(ci >= 1)
            def _():
                scat_wait(lax.rem(ci - 1, 3), nxt)

            @pl.when(ci + 1 < nchunk)
            def _():
                wait_idx(ci + 1, n3)
                gather(ci + 1, n3, nxt)

            @pl.when(ci + 2 < nchunk)
            def _():
                fetch_idx(ci + 2, f3)

            @pl.when(ci >= 2)
            def _():
                wscat_wait(cur)

            for g in range(_CHUNK // _LANES):
                sl = pl.ds(g * _LANES, _LANES)
                dv = dst_c[i3, sl]
                ee = (plsc.load_gather(el_v, [src_c[i3, sl]])
                      + plsc.load_gather(er_v, [dv]))
                ee = jnp.maximum(ee, 0.2 * ee)
                w_v[cur, sl] = jnp.exp(ee - bvec)
                dst_cw[cur, sl] = dv
            pltpu.async_copy(w_v.at[cur], den_sh.at[dst_cw.at[cur]],
                             wsem.at[cur], add=True)

            pltpu.make_async_copy(feat_hbm.at[src_c.at[i3]], rows2.at[cur],
                                  gsem.at[cur]).wait()

            @plsc.parallel_loop(0, _CHUNK, step=1, unroll=8)
            def _(ei):
                wb = plsc.load_gather(
                    w_v.at[cur], [jnp.full((_LANES,), 0, jnp.int32) + ei])
                for db in range(d // _LANES):
                    sl = pl.ds(db * _LANES, _LANES)
                    rows2[cur, ei, sl] = rows2[cur, ei, sl] * wb
            pltpu.async_copy(rows2.at[cur], acc_sh.at[dst_c.at[i3]],
                             ssem.at[cur], add=True)
            return carry

        lax.fori_loop(0, nchunk, chunk_body, 0)
        scat_wait(lax.rem(nchunk - 1, 3), lax.rem(nchunk - 1, 2))
        wscat_wait(lax.rem(nchunk - 1, 2))

        @pl.when(jnp.int32(nchunk) >= 2)
        def _():
            wscat_wait(lax.rem(nchunk - 2, 2))

        plsc.subcore_barrier()

        @pl.when(sid < nslab)
        def _():
            pltpu.sync_copy(acc_sh.at[pl.ds(sid * slab, slab)],
                            acc_out.at[cid, pl.ds(sid * slab, slab)])

        @pl.when(sid == 0)
        def _():
            pltpu.sync_copy(den_sh, den_out.at[cid])

    return sc_edge_kernel


def _combine_body(acc_ref, den_ref, bias_ref, out_ref):
    a = acc_ref[0] + acc_ref[1]
    dcol = jnp.sum(den_ref[...], axis=0)
    safe = jnp.where(dcol > 0.0, a / jnp.where(dcol > 0.0, dcol, 1.0), 0.0)
    out_ref[...] = jnp.tanh(safe + bias_ref[...])


def _combine(acc, den, bias2d):
    nc, n, d = acc.shape
    nt = den.shape[0]
    bn = 2000
    nblk = n // bn
    return pl.pallas_call(
        _combine_body,
        grid=(nblk,),
        in_specs=[
            pl.BlockSpec((nc, bn, d), lambda i: (0, i, 0)),
            pl.BlockSpec((nt, bn, 1), lambda i: (0, i, 0)),
            pl.BlockSpec((1, d), lambda i: (0, 0)),
        ],
        out_specs=pl.BlockSpec((bn, d), lambda i: (i, 0)),
        out_shape=jax.ShapeDtypeStruct((n, d), jnp.float32),
    )(acc, den.reshape(nt, n, 1), bias2d)


def kernel(x, edge_index, W, attn_l, attn_r, bias):
    n, d_in = x.shape
    d = attn_l.shape[1]
    e = edge_index.shape[1]

    feat, el, er, _elm, _erm, boundv = _project(x, W, attn_l, attn_r)

    info = plsc.get_sparse_core_info()
    tiles = info.num_cores * info.num_subcores
    nchunk = e // (tiles * _CHUNK)
    eidx = edge_index.reshape(2 * tiles, nchunk, _CHUNK)
    zrows = jnp.zeros((n, d), jnp.float32)
    zvec = jnp.zeros((n,), jnp.float32)

    sc_kernel = _make_sc_edge_kernel(n, d, e, info.num_cores, info.num_subcores)
    acc, den = sc_kernel(feat, el.reshape(n), er.reshape(n), eidx,
                         boundv, zrows, zvec)

    return _combine(acc, den, bias.reshape(1, d))
